# Initial kernel scaffold; baseline (speedup 1.0000x reference)
import jax
import jax.numpy as jnp
from jax.experimental import pallas as pl


def kernel(x, edge_index, W1, b1, W2, b2):
    raise NotImplementedError("write your pallas kernel here")


# speedup vs baseline: 74.3177x; 74.3177x over previous
import functools

import jax
import jax.numpy as jnp
from jax import lax
from jax.experimental import pallas as pl
from jax.experimental.pallas import tpu as pltpu, tpu_sc as plsc

N = 10000
E = 320000
D = 128
H = 16
A = 5

NC, NS = 2, 16
NW = NC * NS
NP = 10240
ROWS = NP // NS
EPW = E // NW
C = 2000
CH = EPW // C

BR = 2048
NPB = NP // BR

_mesh = plsc.VectorSubcoreMesh(core_axis_name="c", subcore_axis_name="s")
_sc_params = pltpu.CompilerParams(use_tc_tiling_on_sc=False)


@functools.partial(
    pl.kernel,
    mesh=_mesh,
    compiler_params=_sc_params,
    out_type=[jax.ShapeDtypeStruct((A, NC, NP), jnp.float32)],
    scratch_types=[
        pltpu.VMEM((C,), jnp.int32),
        pltpu.VMEM((C,), jnp.float32),
        pltpu.VMEM_SHARED((NP,), jnp.float32),
    ],
)
def _sc_degree(ei_hbm, z1_hbm, ones_hbm, degp_hbm, dbuf, onesbuf, deg_sh):
    c = lax.axis_index("c")
    s = lax.axis_index("s")
    w = s * NC + c
    r0 = s * ROWS
    pltpu.sync_copy(ones_hbm, onesbuf)
    for a in range(A):
        pltpu.sync_copy(z1_hbm.at[pl.ds(r0, ROWS)], deg_sh.at[pl.ds(r0, ROWS)])
        plsc.subcore_barrier()
        for k in range(CH):
            off = w * EPW + k * C
            pltpu.sync_copy(ei_hbm.at[a, 1, pl.ds(off, C)], dbuf)
            pltpu.sync_copy(onesbuf, deg_sh.at[dbuf], add=True)
        plsc.subcore_barrier()
        pltpu.sync_copy(deg_sh.at[pl.ds(r0, ROWS)],
                        degp_hbm.at[a, c, pl.ds(r0, ROWS)])
        plsc.subcore_barrier()


@functools.partial(
    pl.kernel,
    mesh=_mesh,
    compiler_params=_sc_params,
    out_type=[
        jax.ShapeDtypeStruct((A, NC, NP, H), jnp.float32),
        jax.ShapeDtypeStruct((A, NC, NP), jnp.float32),
    ],
    scratch_types=[
        pltpu.VMEM((C,), jnp.int32),
        pltpu.VMEM((C,), jnp.int32),
        pltpu.VMEM((C, H), jnp.float32),
        pltpu.VMEM((C,), jnp.float32),
        pltpu.VMEM_SHARED((NP, H), jnp.float32),
        pltpu.VMEM_SHARED((NP,), jnp.float32),
        pltpu.VMEM_SHARED((NP,), jnp.float32),
        pltpu.SemaphoreType.DMA,
    ],
)
def _sc_edges(ei_hbm, g_hbm, dinv_hbm, z2_hbm, z1_hbm,
              accp_hbm, tp_hbm,
              sbuf, dbuf, rowbuf, valsbuf, acc_sh, t_sh, dinv_sh, sem):
    c = lax.axis_index("c")
    s = lax.axis_index("s")
    w = s * NC + c
    r0 = s * ROWS
    for a in range(A):
        pltpu.sync_copy(z2_hbm.at[pl.ds(r0, ROWS)], acc_sh.at[pl.ds(r0, ROWS)])
        pltpu.sync_copy(z1_hbm.at[pl.ds(r0, ROWS)], t_sh.at[pl.ds(r0, ROWS)])
        pltpu.sync_copy(dinv_hbm.at[a, pl.ds(r0, ROWS)],
                        dinv_sh.at[pl.ds(r0, ROWS)])
        plsc.subcore_barrier()
        for k in range(CH):
            off = w * EPW + k * C
            pltpu.sync_copy(ei_hbm.at[a, 0, pl.ds(off, C)], sbuf)
            pltpu.sync_copy(ei_hbm.at[a, 1, pl.ds(off, C)], dbuf)
            gcp = pltpu.async_copy(g_hbm.at[a].at[sbuf], rowbuf, sem)
            pltpu.sync_copy(dinv_sh.at[dbuf], valsbuf)
            gcp.wait()
            pltpu.sync_copy(rowbuf, acc_sh.at[dbuf], add=True)
            pltpu.sync_copy(valsbuf, t_sh.at[sbuf], add=True)
        plsc.subcore_barrier()
        pltpu.sync_copy(acc_sh.at[pl.ds(r0, ROWS)],
                        accp_hbm.at[a, c, pl.ds(r0, ROWS)])
        pltpu.sync_copy(t_sh.at[pl.ds(r0, ROWS)],
                        tp_hbm.at[a, c, pl.ds(r0, ROWS)])
        plsc.subcore_barrier()


def _tc_g_body(x_ref, w1_ref, degp_ref, g_ref, dinv_ref):
    h = jnp.dot(x_ref[0], w1_ref[...], preferred_element_type=jnp.float32)
    deg = degp_ref[0, 0] + degp_ref[0, 1] + 1.0
    di = lax.rsqrt(deg)
    dinv_ref[0] = di
    g_ref[0] = h * di


def _tc_g(x, w1, degp4):
    return pl.pallas_call(
        _tc_g_body,
        grid=(A, NPB),
        in_specs=[
            pl.BlockSpec((1, BR, D), lambda a, j: (a, j, 0)),
            pl.BlockSpec((D, H), lambda a, j: (0, 0)),
            pl.BlockSpec((1, NC, BR, 1), lambda a, j: (a, 0, j, 0)),
        ],
        out_specs=[
            pl.BlockSpec((1, BR, H), lambda a, j: (a, j, 0)),
            pl.BlockSpec((1, BR, 1), lambda a, j: (a, j, 0)),
        ],
        out_shape=[
            jax.ShapeDtypeStruct((A, NP, H), jnp.float32),
            jax.ShapeDtypeStruct((A, NP, 1), jnp.float32),
        ],
    )(x, w1, degp4)


def _tc_red_body(accp_ref, g_ref, dinv_ref, tp_ref, b1_ref, u_ref):
    j = pl.program_id(1)
    acc = accp_ref[0, 0] + accp_ref[0, 1]
    gb = g_ref[0]
    di = dinv_ref[0]
    t = tp_ref[0, 0] + tp_ref[0, 1]
    out1 = di * (acc + gb) + b1_ref[0][None, :]
    r = jnp.maximum(out1, 0.0)
    cvec = di * (t + di)
    rows = j * BR + lax.broadcasted_iota(jnp.int32, (BR, 1), 0)
    contrib = jnp.where(rows < N, cvec * r, 0.0)
    pu = jnp.sum(contrib, axis=0, keepdims=True)

    @pl.when(j == 0)
    def _():
        u_ref[0] = pu

    @pl.when(j > 0)
    def _():
        u_ref[0] += pu


def _tc_red(accp, g, dinv, tp4, b1):
    return pl.pallas_call(
        _tc_red_body,
        grid=(A, NPB),
        in_specs=[
            pl.BlockSpec((1, NC, BR, H), lambda a, j: (a, 0, j, 0)),
            pl.BlockSpec((1, BR, H), lambda a, j: (a, j, 0)),
            pl.BlockSpec((1, BR, 1), lambda a, j: (a, j, 0)),
            pl.BlockSpec((1, NC, BR, 1), lambda a, j: (a, 0, j, 0)),
            pl.BlockSpec((1, H), lambda a, j: (0, 0)),
        ],
        out_specs=pl.BlockSpec((1, 1, H), lambda a, j: (a, 0, 0)),
        out_shape=jax.ShapeDtypeStruct((A, 1, H), jnp.float32),
    )(accp, g, dinv, tp4, b1)


def _tc_fin_body(u_ref, w2_ref, out_ref):
    uu = u_ref[...]
    w2 = w2_ref[...]
    logits = jnp.sum(uu * w2[:, 0][None, :], axis=1, keepdims=True) / N
    m = jnp.max(logits)
    e = jnp.exp(logits - m)
    out_ref[...] = e / jnp.sum(e)


def _tc_fin(u, w2):
    return pl.pallas_call(
        _tc_fin_body,
        out_shape=jax.ShapeDtypeStruct((A, 1), jnp.float32),
    )(u, w2)


def kernel(x, edge_index, W1, b1, W2, b2):
    ei = edge_index.astype(jnp.int32)
    z1 = jnp.zeros((NP,), jnp.float32)
    z2 = jnp.zeros((NP, H), jnp.float32)
    ones = jnp.ones((C,), jnp.float32)

    (degp,) = _sc_degree(ei, z1, ones)
    g, dinv = _tc_g(x, W1, degp.reshape(A, NC, NP, 1))
    accp, tp = _sc_edges(ei, g, dinv.reshape(A, NP), z2, z1)
    u = _tc_red(accp, g, dinv, tp.reshape(A, NC, NP, 1), b1.reshape(1, H))
    probs = _tc_fin(u.reshape(A, H), W2)
    return probs.reshape(A)


# speedup vs baseline: 83.7047x; 1.1263x over previous
import functools

import jax
import jax.numpy as jnp
from jax import lax
from jax.experimental import pallas as pl
from jax.experimental.pallas import tpu as pltpu, tpu_sc as plsc

N = 10000
E = 320000
D = 128
H = 16
A = 5

NC, NS = 2, 16
NW = NC * NS
NP = 10240
ROWS = NP // NS
EPW = E // NW
C = 2000
CH = EPW // C

BR = 2048
NPB = NP // BR

_mesh = plsc.VectorSubcoreMesh(core_axis_name="c", subcore_axis_name="s")
_sc_params = pltpu.CompilerParams(use_tc_tiling_on_sc=False)


@functools.partial(
    pl.kernel,
    mesh=_mesh,
    compiler_params=_sc_params,
    out_type=[jax.ShapeDtypeStruct((A, NC, NP), jnp.float32)],
    scratch_types=[
        pltpu.VMEM((C,), jnp.int32),
        pltpu.VMEM((C,), jnp.int32),
        pltpu.VMEM((C,), jnp.int32),
        pltpu.VMEM((C,), jnp.int32),
        pltpu.VMEM((C,), jnp.float32),
        pltpu.VMEM_SHARED((NP,), jnp.float32),
        pltpu.SemaphoreType.DMA,
        pltpu.SemaphoreType.DMA,
        pltpu.SemaphoreType.DMA,
        pltpu.SemaphoreType.DMA,
        pltpu.SemaphoreType.DMA,
        pltpu.SemaphoreType.DMA,
        pltpu.SemaphoreType.DMA,
    ],
)
def _sc_degree(ei_hbm, z1_hbm, ones_hbm, degp_hbm,
               db0, db1, db2, db3, onesbuf, deg_sh,
               si0, si1, si2, si3, ss0, ss1, semz):
    c = lax.axis_index("c")
    s = lax.axis_index("s")
    w = s * NC + c
    r0 = s * ROWS
    rs = pl.ds(r0, ROWS)
    dbuf = [db0, db1, db2, db3]
    semi = [si0, si1, si2, si3]
    del ss0, ss1
    pltpu.sync_copy(ones_hbm, onesbuf)
    for a in range(A):
        pltpu.async_copy(z1_hbm.at[rs], deg_sh.at[rs], semz).wait()
        plsc.subcore_barrier()
        icp = {}

        def start_idx(k, a=a):
            b = k % 4
            off = w * EPW + k * C
            icp[k] = pltpu.async_copy(
                ei_hbm.at[a, 1, pl.ds(off, C)], dbuf[b], semi[b])

        start_idx(0)
        start_idx(1)
        for k in range(CH):
            if k + 2 < CH:
                start_idx(k + 2)
            icp[k].wait()
            pltpu.sync_copy(onesbuf, deg_sh.at[dbuf[k % 4]], add=True)
        plsc.subcore_barrier()
        pltpu.async_copy(deg_sh.at[rs], degp_hbm.at[a, c, rs], semz).wait()
        plsc.subcore_barrier()


@functools.partial(
    pl.kernel,
    mesh=_mesh,
    compiler_params=_sc_params,
    out_type=[
        jax.ShapeDtypeStruct((A, NC, NP, H), jnp.float32),
        jax.ShapeDtypeStruct((A, NC, NP), jnp.float32),
    ],
    scratch_types=[
        pltpu.VMEM((CH, C), jnp.int32),
        pltpu.VMEM((CH, C), jnp.int32),
        pltpu.VMEM((C, H), jnp.float32),
        pltpu.VMEM((C, H), jnp.float32),
        pltpu.VMEM((C,), jnp.float32),
        pltpu.VMEM_SHARED((NP, H), jnp.float32),
        pltpu.VMEM_SHARED((NP,), jnp.float32),
        pltpu.VMEM_SHARED((NP,), jnp.float32),
        pltpu.SemaphoreType.DMA,
        pltpu.SemaphoreType.DMA,
        pltpu.SemaphoreType.DMA,
    ],
)
def _sc_edges(ei_hbm, g_hbm, dinv_hbm, z2_hbm, z1_hbm,
              accp_hbm, tp_hbm,
              sbuf, dbuf, rb0, rb1, valsbuf, acc_sh, t_sh, dinv_sh,
              sg0, sg1, semz):
    c = lax.axis_index("c")
    s = lax.axis_index("s")
    w = s * NC + c
    r0 = s * ROWS
    rs = pl.ds(r0, ROWS)
    rowbuf = [rb0, rb1]
    semg = [sg0, sg1]
    for a in range(A):
        zcs = [
            pltpu.async_copy(z2_hbm.at[rs], acc_sh.at[rs], semz),
            pltpu.async_copy(z1_hbm.at[rs], t_sh.at[rs], semz),
            pltpu.async_copy(dinv_hbm.at[a, rs], dinv_sh.at[rs], semz),
        ]
        pltpu.sync_copy(ei_hbm.at[a, 0, w], sbuf)
        pltpu.sync_copy(ei_hbm.at[a, 1, w], dbuf)
        for cp in zcs:
            cp.wait()
        plsc.subcore_barrier()

        gcp = {}

        def start_gather(k, a=a):
            br = k % 2
            gcp[k] = pltpu.async_copy(g_hbm.at[a].at[sbuf.at[k]],
                                      rowbuf[br], semg[br])

        start_gather(0)
        for k in range(CH):
            if k + 1 < CH:
                start_gather(k + 1)
            gcp[k].wait()
            pltpu.sync_copy(dinv_sh.at[dbuf.at[k]], valsbuf)
            pltpu.sync_copy(rowbuf[k % 2], acc_sh.at[dbuf.at[k]], add=True)
            pltpu.sync_copy(valsbuf, t_sh.at[sbuf.at[k]], add=True)
        plsc.subcore_barrier()
        ocs = [
            pltpu.async_copy(acc_sh.at[rs], accp_hbm.at[a, c, rs], semz),
            pltpu.async_copy(t_sh.at[rs], tp_hbm.at[a, c, rs], semz),
        ]
        for cp in ocs:
            cp.wait()
        plsc.subcore_barrier()


def _tc_g_body(x_ref, w1_ref, degp_ref, g_ref, dinv_ref):
    h = jnp.dot(x_ref[0], w1_ref[...], preferred_element_type=jnp.float32)
    deg = degp_ref[0, 0] + degp_ref[0, 1] + 1.0
    di = lax.rsqrt(deg)
    dinv_ref[0] = di
    g_ref[0] = h * di


def _tc_g(x, w1, degp4):
    return pl.pallas_call(
        _tc_g_body,
        grid=(A, NPB),
        in_specs=[
            pl.BlockSpec((1, BR, D), lambda a, j: (a, j, 0)),
            pl.BlockSpec((D, H), lambda a, j: (0, 0)),
            pl.BlockSpec((1, NC, BR, 1), lambda a, j: (a, 0, j, 0)),
        ],
        out_specs=[
            pl.BlockSpec((1, BR, H), lambda a, j: (a, j, 0)),
            pl.BlockSpec((1, BR, 1), lambda a, j: (a, j, 0)),
        ],
        out_shape=[
            jax.ShapeDtypeStruct((A, NP, H), jnp.float32),
            jax.ShapeDtypeStruct((A, NP, 1), jnp.float32),
        ],
    )(x, w1, degp4)


def _tc_red_body(accp_ref, g_ref, dinv_ref, tp_ref, b1_ref, u_ref):
    j = pl.program_id(1)
    acc = accp_ref[0, 0] + accp_ref[0, 1]
    gb = g_ref[0]
    di = dinv_ref[0]
    t = tp_ref[0, 0] + tp_ref[0, 1]
    out1 = di * (acc + gb) + b1_ref[0][None, :]
    r = jnp.maximum(out1, 0.0)
    cvec = di * (t + di)
    rows = j * BR + lax.broadcasted_iota(jnp.int32, (BR, 1), 0)
    contrib = jnp.where(rows < N, cvec * r, 0.0)
    pu = jnp.sum(contrib, axis=0, keepdims=True)

    @pl.when(j == 0)
    def _():
        u_ref[0] = pu

    @pl.when(j > 0)
    def _():
        u_ref[0] += pu


def _tc_red(accp, g, dinv, tp4, b1):
    return pl.pallas_call(
        _tc_red_body,
        grid=(A, NPB),
        in_specs=[
            pl.BlockSpec((1, NC, BR, H), lambda a, j: (a, 0, j, 0)),
            pl.BlockSpec((1, BR, H), lambda a, j: (a, j, 0)),
            pl.BlockSpec((1, BR, 1), lambda a, j: (a, j, 0)),
            pl.BlockSpec((1, NC, BR, 1), lambda a, j: (a, 0, j, 0)),
            pl.BlockSpec((1, H), lambda a, j: (0, 0)),
        ],
        out_specs=pl.BlockSpec((1, 1, H), lambda a, j: (a, 0, 0)),
        out_shape=jax.ShapeDtypeStruct((A, 1, H), jnp.float32),
    )(accp, g, dinv, tp4, b1)


def _tc_fin_body(u_ref, w2_ref, out_ref):
    uu = u_ref[...]
    w2 = w2_ref[...]
    logits = jnp.sum(uu * w2[:, 0][None, :], axis=1, keepdims=True) / N
    m = jnp.max(logits)
    e = jnp.exp(logits - m)
    out_ref[...] = e / jnp.sum(e)


def _tc_fin(u, w2):
    return pl.pallas_call(
        _tc_fin_body,
        out_shape=jax.ShapeDtypeStruct((A, 1), jnp.float32),
    )(u, w2)


def kernel(x, edge_index, W1, b1, W2, b2):
    ei = edge_index.astype(jnp.int32)
    z1 = jnp.zeros((NP,), jnp.float32)
    z2 = jnp.zeros((NP, H), jnp.float32)
    ones = jnp.ones((C,), jnp.float32)

    (degp,) = _sc_degree(ei, z1, ones)
    g, dinv = _tc_g(x, W1, degp.reshape(A, NC, NP, 1))
    accp, tp = _sc_edges(ei.reshape(A, 2, NW, CH, C), g,
                         dinv.reshape(A, NP), z2, z1)
    u = _tc_red(accp, g, dinv, tp.reshape(A, NC, NP, 1), b1.reshape(1, H))
    probs = _tc_fin(u.reshape(A, H), W2)
    return probs.reshape(A)


# speedup vs baseline: 86.2995x; 1.0310x over previous
import functools

import jax
import jax.numpy as jnp
from jax import lax
from jax.experimental import pallas as pl
from jax.experimental.pallas import tpu as pltpu, tpu_sc as plsc

N = 10000
E = 320000
D = 128
H = 16
A = 5

NC, NS = 2, 16
NW = NC * NS
NP = 10240
ROWS = NP // NS
EPW = E // NW
C = 2000
CH = EPW // C

BR = 2048
NPB = NP // BR

_mesh = plsc.VectorSubcoreMesh(core_axis_name="c", subcore_axis_name="s")
_sc_params = pltpu.CompilerParams(use_tc_tiling_on_sc=False)


@functools.partial(
    pl.kernel,
    mesh=_mesh,
    compiler_params=_sc_params,
    out_type=[jax.ShapeDtypeStruct((A, NC, NP), jnp.float32)],
    scratch_types=[
        pltpu.VMEM((EPW,), jnp.int32),
        pltpu.VMEM((EPW,), jnp.int32),
        pltpu.VMEM((EPW,), jnp.float32),
        [pltpu.VMEM_SHARED((NP,), jnp.float32) for _ in range(A)],
        pltpu.SemaphoreType.DMA,
        pltpu.SemaphoreType.DMA,
        pltpu.SemaphoreType.DMA,
    ],
)
def _sc_degree(ei_hbm, z1_hbm, ones_hbm, degp_hbm,
               db0, db1, onesbuf, deg_sh,
               si0, si1, semz):
    c = lax.axis_index("c")
    s = lax.axis_index("s")
    w = s * NC + c
    r0 = s * ROWS
    rs = pl.ds(r0, ROWS)
    dbuf = [db0, db1]
    semi = [si0, si1]
    pltpu.sync_copy(ones_hbm, onesbuf)
    zcs = [pltpu.async_copy(z1_hbm.at[rs], deg_sh[a].at[rs], semz)
           for a in range(A)]
    icp = {a: pltpu.async_copy(ei_hbm.at[a, 1, w], dbuf[a % 2], semi[a % 2])
           for a in range(2)}
    for cp in zcs:
        cp.wait()
    plsc.subcore_barrier()
    for a in range(A):
        icp[a].wait()
        pltpu.sync_copy(onesbuf, deg_sh[a].at[dbuf[a % 2]], add=True)
        if a + 2 < A:
            icp[a + 2] = pltpu.async_copy(ei_hbm.at[a + 2, 1, w],
                                          dbuf[a % 2], semi[a % 2])
    plsc.subcore_barrier()
    ocs = [pltpu.async_copy(deg_sh[a].at[rs], degp_hbm.at[a, c, rs], semz)
           for a in range(A)]
    for cp in ocs:
        cp.wait()


@functools.partial(
    pl.kernel,
    mesh=_mesh,
    compiler_params=_sc_params,
    out_type=[
        jax.ShapeDtypeStruct((A, NC, NP, H), jnp.float32),
        jax.ShapeDtypeStruct((A, NC, NP), jnp.float32),
    ],
    scratch_types=[
        pltpu.VMEM((2, CH, C), jnp.int32),
        pltpu.VMEM((2, CH, C), jnp.int32),
        pltpu.VMEM((C, H), jnp.float32),
        pltpu.VMEM((C, H), jnp.float32),
        pltpu.VMEM((C,), jnp.float32),
        [pltpu.VMEM_SHARED((NP, H), jnp.float32) for _ in range(2)],
        [pltpu.VMEM_SHARED((NP,), jnp.float32) for _ in range(2)],
        [pltpu.VMEM_SHARED((NP,), jnp.float32) for _ in range(2)],
        pltpu.SemaphoreType.DMA,
        pltpu.SemaphoreType.DMA,
        pltpu.SemaphoreType.DMA,
        pltpu.SemaphoreType.DMA,
        pltpu.SemaphoreType.DMA,
    ],
)
def _sc_edges(ei_hbm, g_hbm, dinv_hbm, z2_hbm, z1_hbm,
              accp_hbm, tp_hbm,
              sbuf, dbuf, rb0, rb1, valsbuf, acc_sh, t_sh, dinv_sh,
              sg0, sg1, si0, si1, semz):
    c = lax.axis_index("c")
    s = lax.axis_index("s")
    w = s * NC + c
    r0 = s * ROWS
    rs = pl.ds(r0, ROWS)
    rowbuf = [rb0, rb1]
    semg = [sg0, sg1]
    semi = [si0, si1]
    gcp = {}

    def start_gather(a, k):
        br = (a * CH + k) % 2
        gcp[(a, k)] = pltpu.async_copy(
            g_hbm.at[a].at[sbuf.at[a % 2, k]], rowbuf[br], semg[br])

    for bstart, bend in ((0, 2), (2, 4), (4, A)):
        ans = list(range(bstart, bend))
        zcs = []
        for i, a in enumerate(ans):
            zcs.append(pltpu.async_copy(z2_hbm.at[rs], acc_sh[i].at[rs],
                                        semz))
            zcs.append(pltpu.async_copy(z1_hbm.at[rs], t_sh[i].at[rs], semz))
            zcs.append(pltpu.async_copy(dinv_hbm.at[a, rs],
                                        dinv_sh[i].at[rs], semz))
        icp = {a: (pltpu.async_copy(ei_hbm.at[a, 0, w], sbuf.at[a % 2],
                                    semi[a % 2]),
                   pltpu.async_copy(ei_hbm.at[a, 1, w], dbuf.at[a % 2],
                                    semi[a % 2]))
               for a in ans[:2]}
        for cp in zcs:
            cp.wait()
        plsc.subcore_barrier()
        for cp in icp[ans[0]]:
            cp.wait()
        start_gather(ans[0], 0)
        for i, a in enumerate(ans):
            for k in range(CH):
                if k + 1 < CH:
                    start_gather(a, k + 1)
                elif a + 1 < bend:
                    for cp in icp[a + 1]:
                        cp.wait()
                    start_gather(a + 1, 0)
                gcp[(a, k)].wait()
                pltpu.sync_copy(dinv_sh[i].at[dbuf.at[a % 2, k]], valsbuf)
                pltpu.sync_copy(rowbuf[(a * CH + k) % 2],
                                acc_sh[i].at[dbuf.at[a % 2, k]], add=True)
                pltpu.sync_copy(valsbuf, t_sh[i].at[sbuf.at[a % 2, k]],
                                add=True)
            if a + 2 < bend:
                icp[a + 2] = (
                    pltpu.async_copy(ei_hbm.at[a + 2, 0, w], sbuf.at[a % 2],
                                     semi[a % 2]),
                    pltpu.async_copy(ei_hbm.at[a + 2, 1, w], dbuf.at[a % 2],
                                     semi[a % 2]),
                )
        plsc.subcore_barrier()
        ocs = []
        for i, a in enumerate(ans):
            ocs.append(pltpu.async_copy(acc_sh[i].at[rs],
                                        accp_hbm.at[a, c, rs], semz))
            ocs.append(pltpu.async_copy(t_sh[i].at[rs],
                                        tp_hbm.at[a, c, rs], semz))
        for cp in ocs:
            cp.wait()
        plsc.subcore_barrier()


def _tc_g_body(x_ref, w1_ref, degp_ref, g_ref, dinv_ref):
    h = jnp.dot(x_ref[0], w1_ref[...], preferred_element_type=jnp.float32)
    deg = degp_ref[0, 0] + degp_ref[0, 1] + 1.0
    di = lax.rsqrt(deg)
    dinv_ref[0] = di
    g_ref[0] = h * di


def _tc_g(x, w1, degp4):
    return pl.pallas_call(
        _tc_g_body,
        grid=(A, NPB),
        in_specs=[
            pl.BlockSpec((1, BR, D), lambda a, j: (a, j, 0)),
            pl.BlockSpec((D, H), lambda a, j: (0, 0)),
            pl.BlockSpec((1, NC, BR, 1), lambda a, j: (a, 0, j, 0)),
        ],
        out_specs=[
            pl.BlockSpec((1, BR, H), lambda a, j: (a, j, 0)),
            pl.BlockSpec((1, BR, 1), lambda a, j: (a, j, 0)),
        ],
        out_shape=[
            jax.ShapeDtypeStruct((A, NP, H), jnp.float32),
            jax.ShapeDtypeStruct((A, NP, 1), jnp.float32),
        ],
    )(x, w1, degp4)


def _tc_red_body(accp_ref, g_ref, dinv_ref, tp_ref, b1_ref, u_ref):
    j = pl.program_id(1)
    acc = accp_ref[0, 0] + accp_ref[0, 1]
    gb = g_ref[0]
    di = dinv_ref[0]
    t = tp_ref[0, 0] + tp_ref[0, 1]
    out1 = di * (acc + gb) + b1_ref[0][None, :]
    r = jnp.maximum(out1, 0.0)
    cvec = di * (t + di)
    rows = j * BR + lax.broadcasted_iota(jnp.int32, (BR, 1), 0)
    contrib = jnp.where(rows < N, cvec * r, 0.0)
    pu = jnp.sum(contrib, axis=0, keepdims=True)

    @pl.when(j == 0)
    def _():
        u_ref[0] = pu

    @pl.when(j > 0)
    def _():
        u_ref[0] += pu


def _tc_red(accp, g, dinv, tp4, b1):
    return pl.pallas_call(
        _tc_red_body,
        grid=(A, NPB),
        in_specs=[
            pl.BlockSpec((1, NC, BR, H), lambda a, j: (a, 0, j, 0)),
            pl.BlockSpec((1, BR, H), lambda a, j: (a, j, 0)),
            pl.BlockSpec((1, BR, 1), lambda a, j: (a, j, 0)),
            pl.BlockSpec((1, NC, BR, 1), lambda a, j: (a, 0, j, 0)),
            pl.BlockSpec((1, H), lambda a, j: (0, 0)),
        ],
        out_specs=pl.BlockSpec((1, 1, H), lambda a, j: (a, 0, 0)),
        out_shape=jax.ShapeDtypeStruct((A, 1, H), jnp.float32),
    )(accp, g, dinv, tp4, b1)


def _tc_fin_body(u_ref, w2_ref, out_ref):
    uu = u_ref[:, 0, :]
    w2 = w2_ref[...]
    logits = jnp.sum(uu * w2[:, 0][None, :], axis=1, keepdims=True) / N
    m = jnp.max(logits)
    e = jnp.exp(logits - m)
    out_ref[...] = e / jnp.sum(e)


def _tc_fin(u, w2):
    return pl.pallas_call(
        _tc_fin_body,
        out_shape=jax.ShapeDtypeStruct((A, 1), jnp.float32),
    )(u, w2)


def kernel(x, edge_index, W1, b1, W2, b2):
    ei = edge_index.astype(jnp.int32)
    z1 = jnp.zeros((NP,), jnp.float32)
    z2 = jnp.zeros((NP, H), jnp.float32)
    ones = jnp.ones((EPW,), jnp.float32)

    (degp,) = _sc_degree(ei.reshape(A, 2, NW, EPW), z1, ones)
    g, dinv = _tc_g(x, W1, degp.reshape(A, NC, NP, 1))
    accp, tp = _sc_edges(ei.reshape(A, 2, NW, CH, C), g,
                         dinv.reshape(A, NP), z2, z1)
    u = _tc_red(accp, g, dinv, tp.reshape(A, NC, NP, 1), b1.reshape(1, H))
    probs = _tc_fin(u, W2)
    return probs.reshape(A)


# speedup vs baseline: 87.1227x; 1.0095x over previous
import functools

import jax
import jax.numpy as jnp
from jax import lax
from jax.experimental import pallas as pl
from jax.experimental.pallas import tpu as pltpu, tpu_sc as plsc

N = 10000
E = 320000
D = 128
H = 16
A = 5

NC, NS = 2, 16
NW = NC * NS
NP = 10240
ROWS = NP // NS
EPW = E // NW
C = 2000
CH = EPW // C

BR = 5120
NPB = NP // BR

_mesh = plsc.VectorSubcoreMesh(core_axis_name="c", subcore_axis_name="s")
_sc_params = pltpu.CompilerParams(use_tc_tiling_on_sc=False)


@functools.partial(
    pl.kernel,
    mesh=_mesh,
    compiler_params=_sc_params,
    out_type=[jax.ShapeDtypeStruct((A, NC, NP), jnp.float32)],
    scratch_types=[
        pltpu.VMEM((EPW,), jnp.int32),
        pltpu.VMEM((EPW,), jnp.int32),
        pltpu.VMEM((EPW,), jnp.float32),
        [pltpu.VMEM_SHARED((NP,), jnp.float32) for _ in range(A)],
        pltpu.SemaphoreType.DMA,
        pltpu.SemaphoreType.DMA,
        pltpu.SemaphoreType.DMA,
    ],
)
def _sc_degree(ei_hbm, z1_hbm, ones_hbm, degp_hbm,
               db0, db1, onesbuf, deg_sh,
               si0, si1, semz):
    c = lax.axis_index("c")
    s = lax.axis_index("s")
    w = s * NC + c
    r0 = s * ROWS
    rs = pl.ds(r0, ROWS)
    dbuf = [db0, db1]
    semi = [si0, si1]
    pltpu.sync_copy(ones_hbm, onesbuf)
    zcs = [pltpu.async_copy(z1_hbm.at[rs], deg_sh[a].at[rs], semz)
           for a in range(A)]
    icp = {a: pltpu.async_copy(ei_hbm.at[a, 1, pl.ds(w * EPW, EPW)],
                               dbuf[a % 2], semi[a % 2])
           for a in range(2)}
    for cp in zcs:
        cp.wait()
    plsc.subcore_barrier()
    for a in range(A):
        icp[a].wait()
        pltpu.sync_copy(onesbuf, deg_sh[a].at[dbuf[a % 2]], add=True)
        if a + 2 < A:
            icp[a + 2] = pltpu.async_copy(
                ei_hbm.at[a + 2, 1, pl.ds(w * EPW, EPW)],
                dbuf[a % 2], semi[a % 2])
    plsc.subcore_barrier()
    ocs = [pltpu.async_copy(deg_sh[a].at[rs], degp_hbm.at[a, c, rs], semz)
           for a in range(A)]
    for cp in ocs:
        cp.wait()


@functools.partial(
    pl.kernel,
    mesh=_mesh,
    compiler_params=_sc_params,
    out_type=[
        jax.ShapeDtypeStruct((A, NC, NP, H), jnp.float32),
        jax.ShapeDtypeStruct((A, NC, NP), jnp.float32),
    ],
    scratch_types=[
        pltpu.VMEM((2, CH, C), jnp.int32),
        pltpu.VMEM((2, CH, C), jnp.int32),
        pltpu.VMEM((C, H), jnp.float32),
        pltpu.VMEM((C, H), jnp.float32),
        pltpu.VMEM((C,), jnp.float32),
        [pltpu.VMEM_SHARED((NP, H), jnp.float32) for _ in range(2)],
        [pltpu.VMEM_SHARED((NP,), jnp.float32) for _ in range(2)],
        [pltpu.VMEM_SHARED((NP,), jnp.float32) for _ in range(2)],
        pltpu.SemaphoreType.DMA,
        pltpu.SemaphoreType.DMA,
        pltpu.SemaphoreType.DMA,
        pltpu.SemaphoreType.DMA,
        pltpu.SemaphoreType.DMA,
    ],
)
def _sc_edges(ei_hbm, g_hbm, dinv_hbm, z2_hbm, z1_hbm,
              accp_hbm, tp_hbm,
              sbuf, dbuf, rb0, rb1, valsbuf, acc_sh, t_sh, dinv_sh,
              sg0, sg1, si0, si1, semz):
    c = lax.axis_index("c")
    s = lax.axis_index("s")
    w = s * NC + c
    r0 = s * ROWS
    rs = pl.ds(r0, ROWS)
    rowbuf = [rb0, rb1]
    semg = [sg0, sg1]
    semi = [si0, si1]
    gcp = {}
    icp = {}

    def start_gather(a, k):
        br = (a * CH + k) % 2
        gcp[(a, k)] = pltpu.async_copy(
            g_hbm.at[a].at[sbuf.at[a % 2, k]], rowbuf[br], semg[br])

    def load_idx(a):
        b = a % 2
        cps = []
        for k in range(CH):
            off = w * EPW + k * C
            cps.append(pltpu.async_copy(ei_hbm.at[a, 0, pl.ds(off, C)],
                                        sbuf.at[b, k], semi[b]))
            cps.append(pltpu.async_copy(ei_hbm.at[a, 1, pl.ds(off, C)],
                                        dbuf.at[b, k], semi[b]))
        icp[a] = cps

    for bstart, bend in ((0, 2), (2, 4), (4, A)):
        ans = list(range(bstart, bend))
        zcs = []
        for i, a in enumerate(ans):
            zcs.append(pltpu.async_copy(z2_hbm.at[rs], acc_sh[i].at[rs],
                                        semz))
            zcs.append(pltpu.async_copy(z1_hbm.at[rs], t_sh[i].at[rs], semz))
            zcs.append(pltpu.async_copy(dinv_hbm.at[a, rs],
                                        dinv_sh[i].at[rs], semz))
        for a in ans[:2]:
            load_idx(a)
        for cp in zcs:
            cp.wait()
        plsc.subcore_barrier()
        for cp in icp[ans[0]]:
            cp.wait()
        start_gather(ans[0], 0)
        for i, a in enumerate(ans):
            for k in range(CH):
                if k + 1 < CH:
                    start_gather(a, k + 1)
                elif a + 1 < bend:
                    for cp in icp[a + 1]:
                        cp.wait()
                    start_gather(a + 1, 0)
                gcp[(a, k)].wait()
                pltpu.sync_copy(dinv_sh[i].at[dbuf.at[a % 2, k]], valsbuf)
                pltpu.sync_copy(rowbuf[(a * CH + k) % 2],
                                acc_sh[i].at[dbuf.at[a % 2, k]], add=True)
                pltpu.sync_copy(valsbuf, t_sh[i].at[sbuf.at[a % 2, k]],
                                add=True)
            if a + 2 < bend:
                load_idx(a + 2)
        plsc.subcore_barrier()
        ocs = []
        for i, a in enumerate(ans):
            ocs.append(pltpu.async_copy(acc_sh[i].at[rs],
                                        accp_hbm.at[a, c, rs], semz))
            ocs.append(pltpu.async_copy(t_sh[i].at[rs],
                                        tp_hbm.at[a, c, rs], semz))
        for cp in ocs:
            cp.wait()
        plsc.subcore_barrier()


def _tc_g_body(x_ref, w1_ref, degp_ref, g_ref, dinv_ref):
    h = jnp.dot(x_ref[0], w1_ref[...], preferred_element_type=jnp.float32)
    deg = degp_ref[0, 0] + degp_ref[0, 1] + 1.0
    di = lax.rsqrt(deg)
    dinv_ref[0] = di
    g_ref[0] = h * di


def _tc_g(x, w1, degp4):
    return pl.pallas_call(
        _tc_g_body,
        grid=(A, NPB),
        in_specs=[
            pl.BlockSpec((1, BR, D), lambda a, j: (a, j, 0)),
            pl.BlockSpec((D, H), lambda a, j: (0, 0)),
            pl.BlockSpec((1, NC, BR, 1), lambda a, j: (a, 0, j, 0)),
        ],
        out_specs=[
            pl.BlockSpec((1, BR, H), lambda a, j: (a, j, 0)),
            pl.BlockSpec((1, BR, 1), lambda a, j: (a, j, 0)),
        ],
        out_shape=[
            jax.ShapeDtypeStruct((A, NP, H), jnp.float32),
            jax.ShapeDtypeStruct((A, NP, 1), jnp.float32),
        ],
    )(x, w1, degp4)


def _tc_red_body(accp_ref, g_ref, dinv_ref, tp_ref, b1_ref, u_ref):
    j = pl.program_id(1)
    acc = accp_ref[0, 0] + accp_ref[0, 1]
    gb = g_ref[0]
    di = dinv_ref[0]
    t = tp_ref[0, 0] + tp_ref[0, 1]
    out1 = di * (acc + gb) + b1_ref[0][None, :]
    r = jnp.maximum(out1, 0.0)
    cvec = di * (t + di)
    rows = j * BR + lax.broadcasted_iota(jnp.int32, (BR, 1), 0)
    contrib = jnp.where(rows < N, cvec * r, 0.0)
    pu = jnp.sum(contrib, axis=0, keepdims=True)

    @pl.when(j == 0)
    def _():
        u_ref[0] = pu

    @pl.when(j > 0)
    def _():
        u_ref[0] += pu


def _tc_red(accp, g, dinv, tp4, b1):
    return pl.pallas_call(
        _tc_red_body,
        grid=(A, NPB),
        in_specs=[
            pl.BlockSpec((1, NC, BR, H), lambda a, j: (a, 0, j, 0)),
            pl.BlockSpec((1, BR, H), lambda a, j: (a, j, 0)),
            pl.BlockSpec((1, BR, 1), lambda a, j: (a, j, 0)),
            pl.BlockSpec((1, NC, BR, 1), lambda a, j: (a, 0, j, 0)),
            pl.BlockSpec((1, H), lambda a, j: (0, 0)),
        ],
        out_specs=pl.BlockSpec((1, 1, H), lambda a, j: (a, 0, 0)),
        out_shape=jax.ShapeDtypeStruct((A, 1, H), jnp.float32),
    )(accp, g, dinv, tp4, b1)


def _tc_fin_body(u_ref, w2_ref, out_ref):
    uu = u_ref[:, 0, :]
    w2 = w2_ref[...]
    logits = jnp.sum(uu * w2[:, 0][None, :], axis=1, keepdims=True) / N
    m = jnp.max(logits)
    e = jnp.exp(logits - m)
    out_ref[...] = e / jnp.sum(e)


def _tc_fin(u, w2):
    return pl.pallas_call(
        _tc_fin_body,
        out_shape=jax.ShapeDtypeStruct((A, 1), jnp.float32),
    )(u, w2)


def kernel(x, edge_index, W1, b1, W2, b2):
    ei = edge_index.astype(jnp.int32)
    z1 = jnp.zeros((NP,), jnp.float32)
    z2 = jnp.zeros((NP, H), jnp.float32)
    ones = jnp.ones((EPW,), jnp.float32)

    (degp,) = _sc_degree(ei, z1, ones)
    g, dinv = _tc_g(x, W1, degp.reshape(A, NC, NP, 1))
    accp, tp = _sc_edges(ei, g, dinv.reshape(A, NP), z2, z1)
    u = _tc_red(accp, g, dinv, tp.reshape(A, NC, NP, 1), b1.reshape(1, H))
    probs = _tc_fin(u, W2)
    return probs.reshape(A)


# speedup vs baseline: 133.4181x; 1.5314x over previous
import functools

import jax
import jax.numpy as jnp
from jax import lax
from jax.experimental import pallas as pl
from jax.experimental.pallas import tpu as pltpu, tpu_sc as plsc

N = 10000
E = 320000
D = 128
H = 16
A = 5

NC, NS = 2, 16
NW = NC * NS
NP = 10240
ROWS = NP // NS
EPW = E // NW
C = 2000
CH = EPW // C

BR = 5120
NPB = NP // BR

_mesh = plsc.VectorSubcoreMesh(core_axis_name="c", subcore_axis_name="s")
_sc_params = pltpu.CompilerParams(use_tc_tiling_on_sc=False)


@functools.partial(
    pl.kernel,
    mesh=_mesh,
    compiler_params=_sc_params,
    out_type=[jax.ShapeDtypeStruct((A, NC, NP), jnp.float32)],
    scratch_types=[
        pltpu.VMEM((EPW,), jnp.int32),
        pltpu.VMEM((EPW,), jnp.int32),
        pltpu.VMEM((EPW,), jnp.float32),
        [pltpu.VMEM_SHARED((NP,), jnp.float32) for _ in range(A)],
        pltpu.SemaphoreType.DMA,
        pltpu.SemaphoreType.DMA,
        pltpu.SemaphoreType.DMA,
    ],
)
def _sc_degree(ei_hbm, z1_hbm, ones_hbm, degp_hbm,
               db0, db1, onesbuf, deg_sh,
               si0, si1, semz):
    c = lax.axis_index("c")
    s = lax.axis_index("s")
    w = s * NC + c
    r0 = s * ROWS
    rs = pl.ds(r0, ROWS)
    dbuf = [db0, db1]
    semi = [si0, si1]
    pltpu.sync_copy(ones_hbm, onesbuf)
    zcs = [pltpu.async_copy(z1_hbm.at[rs], deg_sh[a].at[rs], semz)
           for a in range(A)]
    icp = {a: pltpu.async_copy(ei_hbm.at[a, 1, pl.ds(w * EPW, EPW)],
                               dbuf[a % 2], semi[a % 2])
           for a in range(2)}
    for cp in zcs:
        cp.wait()
    plsc.subcore_barrier()
    for a in range(A):
        icp[a].wait()
        pltpu.sync_copy(onesbuf, deg_sh[a].at[dbuf[a % 2]], add=True)
        if a + 2 < A:
            icp[a + 2] = pltpu.async_copy(
                ei_hbm.at[a + 2, 1, pl.ds(w * EPW, EPW)],
                dbuf[a % 2], semi[a % 2])
    plsc.subcore_barrier()
    ocs = [pltpu.async_copy(deg_sh[a].at[rs], degp_hbm.at[a, c, rs], semz)
           for a in range(A)]
    for cp in ocs:
        cp.wait()


@functools.partial(
    pl.kernel,
    mesh=_mesh,
    compiler_params=_sc_params,
    out_type=[
        jax.ShapeDtypeStruct((A, NC, NP, H), jnp.float32),
        jax.ShapeDtypeStruct((A, NC, NP), jnp.float32),
    ],
    scratch_types=[
        pltpu.VMEM((2, CH, C), jnp.int32),
        pltpu.VMEM((2, CH, C), jnp.int32),
        pltpu.VMEM((C, H), jnp.float32),
        pltpu.VMEM((C, H), jnp.float32),
        pltpu.VMEM((C,), jnp.float32),
        [pltpu.VMEM_SHARED((NP, H), jnp.float32) for _ in range(2)],
        [pltpu.VMEM_SHARED((NP,), jnp.float32) for _ in range(2)],
        [pltpu.VMEM_SHARED((NP,), jnp.float32) for _ in range(2)],
        pltpu.SemaphoreType.DMA,
        pltpu.SemaphoreType.DMA,
        pltpu.SemaphoreType.DMA,
        pltpu.SemaphoreType.DMA,
        pltpu.SemaphoreType.DMA,
    ],
)
def _sc_edges(ei_hbm, g_hbm, dinv_hbm, z1_hbm,
              accp_hbm, tp_hbm,
              sbuf, dbuf, rb0, rb1, valsbuf, acc_sh, t_sh, dinv_sh,
              sg0, sg1, si0, si1, semz):
    c = lax.axis_index("c")
    s = lax.axis_index("s")
    w = s * NC + c
    r0 = s * ROWS
    rs = pl.ds(r0, ROWS)
    rowbuf = [rb0, rb1]
    semg = [sg0, sg1]
    semi = [si0, si1]
    gcp = {}
    icp = {}

    def start_gather(a, k):
        br = (a * CH + k) % 2
        gcp[(a, k)] = pltpu.async_copy(
            g_hbm.at[a].at[sbuf.at[a % 2, k]], rowbuf[br], semg[br])

    def load_idx(a):
        b = a % 2
        cps = []
        for k in range(CH):
            off = w * EPW + k * C
            cps.append(pltpu.async_copy(ei_hbm.at[a, 0, pl.ds(off, C)],
                                        sbuf.at[b, k], semi[b]))
            cps.append(pltpu.async_copy(ei_hbm.at[a, 1, pl.ds(off, C)],
                                        dbuf.at[b, k], semi[b]))
        icp[a] = cps

    for bstart, bend in ((0, 2), (2, 4), (4, A)):
        ans = list(range(bstart, bend))
        zcs = []
        for i, a in enumerate(ans):
            zcs.append(pltpu.async_copy(g_hbm.at[a, rs], acc_sh[i].at[rs],
                                        semz))
            zcs.append(pltpu.async_copy(z1_hbm.at[rs], t_sh[i].at[rs], semz))
            zcs.append(pltpu.async_copy(
                dinv_hbm.at[pl.ds(a * NP + r0, ROWS)],
                dinv_sh[i].at[rs], semz))
        for a in ans[:2]:
            load_idx(a)
        for cp in zcs:
            cp.wait()
        plsc.subcore_barrier()
        for cp in icp[ans[0]]:
            cp.wait()
        start_gather(ans[0], 0)
        for i, a in enumerate(ans):
            for k in range(CH):
                if k + 1 < CH:
                    start_gather(a, k + 1)
                elif a + 1 < bend:
                    for cp in icp[a + 1]:
                        cp.wait()
                    start_gather(a + 1, 0)
                gcp[(a, k)].wait()
                pltpu.sync_copy(dinv_sh[i].at[dbuf.at[a % 2, k]], valsbuf)
                pltpu.sync_copy(rowbuf[(a * CH + k) % 2],
                                acc_sh[i].at[dbuf.at[a % 2, k]], add=True)
                pltpu.sync_copy(valsbuf, t_sh[i].at[sbuf.at[a % 2, k]],
                                add=True)
            if a + 2 < bend:
                load_idx(a + 2)
        plsc.subcore_barrier()
        ocs = []
        for i, a in enumerate(ans):
            ocs.append(pltpu.async_copy(acc_sh[i].at[rs],
                                        accp_hbm.at[a, c, rs], semz))
            ocs.append(pltpu.async_copy(t_sh[i].at[rs],
                                        tp_hbm.at[a, c, rs], semz))
        for cp in ocs:
            cp.wait()
        plsc.subcore_barrier()


def _tc_g_body(x_ref, w1_ref, degp_ref, g_ref, dinv_ref):
    h = jnp.dot(x_ref[0], w1_ref[...], preferred_element_type=jnp.float32)
    deg = degp_ref[0, 0] + degp_ref[0, 1] + 1.0
    di = lax.rsqrt(deg)
    dinv_ref[...] = di
    g_ref[0] = h * di[:, None]


def _tc_g(x, w1, degp):
    return pl.pallas_call(
        _tc_g_body,
        grid=(A, NPB),
        in_specs=[
            pl.BlockSpec((1, BR, D), lambda a, j: (a, j, 0)),
            pl.BlockSpec((D, H), lambda a, j: (0, 0)),
            pl.BlockSpec((1, NC, BR), lambda a, j: (a, 0, j)),
        ],
        out_specs=[
            pl.BlockSpec((1, BR, H), lambda a, j: (a, j, 0)),
            pl.BlockSpec((BR,), lambda a, j: (a * NPB + j,)),
        ],
        out_shape=[
            jax.ShapeDtypeStruct((A, NP, H), jnp.float32),
            jax.ShapeDtypeStruct((A * NP,), jnp.float32),
        ],
    )(x, w1, degp)


def _tc_red_body(accp_ref, degp_ref, tp_ref, b1_ref, u_ref):
    j = pl.program_id(1)
    acc = accp_ref[0, 0] + accp_ref[0, 1]
    deg = degp_ref[0, 0] + degp_ref[0, 1] + 1.0
    di = lax.rsqrt(deg)
    t = tp_ref[0, 0] + tp_ref[0, 1]
    out1 = di[:, None] * acc + b1_ref[0][None, :]
    r = jnp.maximum(out1, 0.0)
    cvec = di * (t + di)
    rows = j * BR + lax.broadcasted_iota(jnp.int32, (BR, 1), 0)
    contrib = jnp.where(rows < N, cvec[:, None] * r, 0.0)
    pu = jnp.sum(contrib, axis=0, keepdims=True)

    @pl.when(j == 0)
    def _():
        u_ref[0] = pu

    @pl.when(j > 0)
    def _():
        u_ref[0] += pu


def _tc_red(accp, degp, tp, b1):
    return pl.pallas_call(
        _tc_red_body,
        grid=(A, NPB),
        in_specs=[
            pl.BlockSpec((1, NC, BR, H), lambda a, j: (a, 0, j, 0)),
            pl.BlockSpec((1, NC, BR), lambda a, j: (a, 0, j)),
            pl.BlockSpec((1, NC, BR), lambda a, j: (a, 0, j)),
            pl.BlockSpec((1, H), lambda a, j: (0, 0)),
        ],
        out_specs=pl.BlockSpec((1, 1, H), lambda a, j: (a, 0, 0)),
        out_shape=jax.ShapeDtypeStruct((A, 1, H), jnp.float32),
    )(accp, degp, tp, b1)


def _tc_fin_body(u_ref, w2_ref, out_ref):
    uu = u_ref[:, 0, :]
    w2 = w2_ref[...]
    logits = jnp.sum(uu * w2[:, 0][None, :], axis=1, keepdims=True) / N
    m = jnp.max(logits)
    e = jnp.exp(logits - m)
    out_ref[...] = e / jnp.sum(e)


def _tc_fin(u, w2):
    return pl.pallas_call(
        _tc_fin_body,
        out_shape=jax.ShapeDtypeStruct((A, 1), jnp.float32),
    )(u, w2)


def kernel(x, edge_index, W1, b1, W2, b2):
    ei = edge_index.astype(jnp.int32)
    z1 = jnp.zeros((NP,), jnp.float32)
    z2 = jnp.zeros((NP, H), jnp.float32)
    ones = jnp.ones((EPW,), jnp.float32)

    (degp,) = _sc_degree(ei, z1, ones)
    g, dinv = _tc_g(x, W1, degp)
    accp, tp = _sc_edges(ei, g, dinv, z1)
    u = _tc_red(accp, degp, tp, b1.reshape(1, H))
    probs = _tc_fin(u, W2)
    return probs.reshape(A)


# speedup vs baseline: 134.3905x; 1.0073x over previous
import functools

import jax
import jax.numpy as jnp
from jax import lax
from jax.experimental import pallas as pl
from jax.experimental.pallas import tpu as pltpu, tpu_sc as plsc

N = 10000
E = 320000
D = 128
H = 16
A = 5

NC, NS = 2, 16
NW = NC * NS
NP = 10240
ROWS = NP // NS
EPW = E // NW
C = 2000
CH = EPW // C

BR = 5120
NPB = NP // BR

_mesh = plsc.VectorSubcoreMesh(core_axis_name="c", subcore_axis_name="s")
_sc_params = pltpu.CompilerParams(use_tc_tiling_on_sc=False)


@functools.partial(
    pl.kernel,
    mesh=_mesh,
    compiler_params=_sc_params,
    out_type=[jax.ShapeDtypeStruct((A, NC, NP), jnp.float32)],
    scratch_types=[
        pltpu.VMEM((EPW,), jnp.int32),
        pltpu.VMEM((EPW,), jnp.int32),
        pltpu.VMEM((EPW,), jnp.float32),
        [pltpu.VMEM_SHARED((NP,), jnp.float32) for _ in range(A)],
        pltpu.SemaphoreType.DMA,
        pltpu.SemaphoreType.DMA,
        pltpu.SemaphoreType.DMA,
    ],
)
def _sc_degree(ei_hbm, z1_hbm, ones_hbm, degp_hbm,
               db0, db1, onesbuf, deg_sh,
               si0, si1, semz):
    c = lax.axis_index("c")
    s = lax.axis_index("s")
    w = s * NC + c
    r0 = s * ROWS
    rs = pl.ds(r0, ROWS)
    dbuf = [db0, db1]
    semi = [si0, si1]
    pltpu.sync_copy(ones_hbm, onesbuf)
    zcs = [pltpu.async_copy(z1_hbm.at[rs], deg_sh[a].at[rs], semz)
           for a in range(A)]
    icp = {a: pltpu.async_copy(ei_hbm.at[a, 1, pl.ds(w * EPW, EPW)],
                               dbuf[a % 2], semi[a % 2])
           for a in range(2)}
    for cp in zcs:
        cp.wait()
    plsc.subcore_barrier()
    for a in range(A):
        icp[a].wait()
        pltpu.sync_copy(onesbuf, deg_sh[a].at[dbuf[a % 2]], add=True)
        if a + 2 < A:
            icp[a + 2] = pltpu.async_copy(
                ei_hbm.at[a + 2, 1, pl.ds(w * EPW, EPW)],
                dbuf[a % 2], semi[a % 2])
    plsc.subcore_barrier()
    ocs = [pltpu.async_copy(deg_sh[a].at[rs], degp_hbm.at[a, c, rs], semz)
           for a in range(A)]
    for cp in ocs:
        cp.wait()


@functools.partial(
    pl.kernel,
    mesh=_mesh,
    compiler_params=_sc_params,
    out_type=[
        jax.ShapeDtypeStruct((A, NC, NP, H), jnp.float32),
        jax.ShapeDtypeStruct((A, NC, NP), jnp.float32),
    ],
    scratch_types=[
        pltpu.VMEM((2, CH, C), jnp.int32),
        pltpu.VMEM((2, CH, C), jnp.int32),
        pltpu.VMEM((C, H), jnp.float32),
        pltpu.VMEM((C, H), jnp.float32),
        pltpu.VMEM((C,), jnp.float32),
        [pltpu.VMEM_SHARED((NP, H), jnp.float32) for _ in range(2)],
        [pltpu.VMEM_SHARED((NP,), jnp.float32) for _ in range(2)],
        [pltpu.VMEM_SHARED((NP,), jnp.float32) for _ in range(2)],
        pltpu.SemaphoreType.DMA,
        pltpu.SemaphoreType.DMA,
        pltpu.SemaphoreType.DMA,
        pltpu.SemaphoreType.DMA,
        pltpu.SemaphoreType.DMA,
    ],
)
def _sc_edges(ei_hbm, g_hbm, dinv_hbm, z1_hbm,
              accp_hbm, tp_hbm,
              sbuf, dbuf, rb0, rb1, valsbuf, acc_sh, t_sh, dinv_sh,
              sg0, sg1, si0, si1, semz):
    c = lax.axis_index("c")
    s = lax.axis_index("s")
    w = s * NC + c
    r0 = s * ROWS
    rs = pl.ds(r0, ROWS)
    rowbuf = [rb0, rb1]
    semg = [sg0, sg1]
    semi = [si0, si1]
    gcp = {}
    icp = {}

    def start_gather(a, k):
        br = (a * CH + k) % 2
        gcp[(a, k)] = pltpu.async_copy(
            g_hbm.at[a].at[sbuf.at[a % 2, k]], rowbuf[br], semg[br])

    def load_idx(a):
        b = a % 2
        cps = []
        for k in range(CH):
            off = w * EPW + k * C
            cps.append(pltpu.async_copy(ei_hbm.at[a, 0, pl.ds(off, C)],
                                        sbuf.at[b, k], semi[b]))
            cps.append(pltpu.async_copy(ei_hbm.at[a, 1, pl.ds(off, C)],
                                        dbuf.at[b, k], semi[b]))
        icp[a] = cps

    for bstart, bend in ((0, 2), (2, 4), (4, A)):
        ans = list(range(bstart, bend))
        zcs = []
        for i, a in enumerate(ans):
            zcs.append(pltpu.async_copy(g_hbm.at[a, rs], acc_sh[i].at[rs],
                                        semz))
            zcs.append(pltpu.async_copy(z1_hbm.at[rs], t_sh[i].at[rs], semz))
            zcs.append(pltpu.async_copy(
                dinv_hbm.at[pl.ds(a * NP + r0, ROWS)],
                dinv_sh[i].at[rs], semz))
        for a in ans[:2]:
            load_idx(a)
        for cp in zcs:
            cp.wait()
        plsc.subcore_barrier()
        for cp in icp[ans[0]]:
            cp.wait()
        start_gather(ans[0], 0)
        for i, a in enumerate(ans):
            for k in range(CH):
                if k + 1 < CH:
                    start_gather(a, k + 1)
                elif a + 1 < bend:
                    for cp in icp[a + 1]:
                        cp.wait()
                    start_gather(a + 1, 0)
                gcp[(a, k)].wait()
                pltpu.sync_copy(dinv_sh[i].at[dbuf.at[a % 2, k]], valsbuf)
                pltpu.sync_copy(rowbuf[(a * CH + k) % 2],
                                acc_sh[i].at[dbuf.at[a % 2, k]], add=True)
                pltpu.sync_copy(valsbuf, t_sh[i].at[sbuf.at[a % 2, k]],
                                add=True)
            if a + 2 < bend:
                load_idx(a + 2)
        plsc.subcore_barrier()
        ocs = []
        for i, a in enumerate(ans):
            ocs.append(pltpu.async_copy(acc_sh[i].at[rs],
                                        accp_hbm.at[a, c, rs], semz))
            ocs.append(pltpu.async_copy(t_sh[i].at[rs],
                                        tp_hbm.at[a, c, rs], semz))
        for cp in ocs:
            cp.wait()
        plsc.subcore_barrier()


def _tc_g_body(x_ref, w1_ref, degp_ref, g_ref, dinv_ref):
    h = jnp.dot(x_ref[0], w1_ref[...], preferred_element_type=jnp.float32)
    deg = degp_ref[0, 0] + degp_ref[0, 1] + 1.0
    di = lax.rsqrt(deg)
    dinv_ref[...] = di
    g_ref[0] = h * di[:, None]


def _tc_g(x, w1, degp):
    return pl.pallas_call(
        _tc_g_body,
        grid=(A, NPB),
        in_specs=[
            pl.BlockSpec((1, BR, D), lambda a, j: (a, j, 0)),
            pl.BlockSpec((D, H), lambda a, j: (0, 0)),
            pl.BlockSpec((1, NC, BR), lambda a, j: (a, 0, j)),
        ],
        out_specs=[
            pl.BlockSpec((1, BR, H), lambda a, j: (a, j, 0)),
            pl.BlockSpec((BR,), lambda a, j: (a * NPB + j,)),
        ],
        out_shape=[
            jax.ShapeDtypeStruct((A, NP, H), jnp.float32),
            jax.ShapeDtypeStruct((A * NP,), jnp.float32),
        ],
    )(x, w1, degp)


def _tc_red_body(accp_ref, degp_ref, tp_ref, b1_ref, u_ref):
    j = pl.program_id(1)
    acc = accp_ref[0, 0] + accp_ref[0, 1]
    deg = degp_ref[0, 0] + degp_ref[0, 1] + 1.0
    di = lax.rsqrt(deg)
    t = tp_ref[0, 0] + tp_ref[0, 1]
    out1 = di[:, None] * acc + b1_ref[0][None, :]
    r = jnp.maximum(out1, 0.0)
    cvec = di * (t + di)
    rows = j * BR + lax.broadcasted_iota(jnp.int32, (BR, 1), 0)
    contrib = jnp.where(rows < N, cvec[:, None] * r, 0.0)
    pu = jnp.sum(contrib, axis=0, keepdims=True)

    @pl.when(j == 0)
    def _():
        u_ref[0] = pu

    @pl.when(j > 0)
    def _():
        u_ref[0] += pu


def _tc_red(accp, degp, tp, b1):
    return pl.pallas_call(
        _tc_red_body,
        grid=(A, NPB),
        in_specs=[
            pl.BlockSpec((1, NC, BR, H), lambda a, j: (a, 0, j, 0)),
            pl.BlockSpec((1, NC, BR), lambda a, j: (a, 0, j)),
            pl.BlockSpec((1, NC, BR), lambda a, j: (a, 0, j)),
            pl.BlockSpec((1, H), lambda a, j: (0, 0)),
        ],
        out_specs=pl.BlockSpec((1, 1, H), lambda a, j: (a, 0, 0)),
        out_shape=jax.ShapeDtypeStruct((A, 1, H), jnp.float32),
    )(accp, degp, tp, b1)


def _tc_fin_body(u_ref, w2_ref, out_ref):
    uu = u_ref[:, 0, :]
    w2 = w2_ref[...]
    logits = jnp.sum(uu * w2[:, 0][None, :], axis=1, keepdims=True) / N
    m = jnp.max(logits)
    e = jnp.exp(logits - m)
    out_ref[...] = e / jnp.sum(e)


def _tc_fin(u, w2):
    return pl.pallas_call(
        _tc_fin_body,
        out_shape=jax.ShapeDtypeStruct((A, 1), jnp.float32),
    )(u, w2)


def kernel(x, edge_index, W1, b1, W2, b2):
    ei = edge_index.astype(jnp.int32)
    z1 = jnp.zeros((NP,), jnp.float32)
    ones = jnp.ones((EPW,), jnp.float32)

    (degp,) = _sc_degree(ei, z1, ones)
    g, dinv = _tc_g(x, W1, degp)
    accp, tp = _sc_edges(ei, g, dinv, z1)
    u = _tc_red(accp, degp, tp, b1.reshape(1, H))
    probs = _tc_fin(u, W2)
    return probs.reshape(A)
